# Initial kernel scaffold; baseline (speedup 1.0000x reference)
import jax
import jax.numpy as jnp
from jax.experimental import pallas as pl


def kernel(x, W1, b1, W2, b2, W3, b3):
    raise NotImplementedError("write your pallas kernel here")


# speedup vs baseline: 106.4705x; 106.4705x over previous
import functools

import jax
import jax.numpy as jnp
from jax.experimental import pallas as pl


def _ode_body(x_ref, w1_ref, b1_ref, w2_ref, b2_ref, w3_ref, b3_ref, out_ref,
              *, num_steps):
    dt = jnp.float32(0.1)
    w1 = w1_ref[...]
    b1 = b1_ref[...]
    w2 = w2_ref[...]
    b2 = b2_ref[...]

    def f(y):
        h = jnp.dot(y, w1, preferred_element_type=jnp.float32) + b1
        h = jnp.maximum(h, 0.0)
        return jnp.dot(h, w2, preferred_element_type=jnp.float32) + b2

    def step(_, y):
        k1 = f(y)
        k2 = f(y + dt * k1 / 3.0)
        k3 = f(y + dt * (k2 - k1 / 3.0))
        k4 = f(y + dt * (k1 - k2 + k3))
        return y + dt * (k1 + 3.0 * (k2 + k3) + k4) / 8.0

    y = jax.lax.fori_loop(0, num_steps, step, x_ref[...])
    out_ref[...] = (jnp.dot(y, w3_ref[...], preferred_element_type=jnp.float32)
                    + b3_ref[...])


@jax.jit
def kernel(x, W1, b1, W2, b2, W3, b3):
    n, in_c = x.shape
    hid = W1.shape[0]
    out_c = W3.shape[0]

    block = 2000
    if n % block != 0:
        block = next(b for b in (1000, 500, 200, 100, 8, 1) if n % b == 0)
    grid = (n // block,)

    full = lambda i: (0, 0)
    out = pl.pallas_call(
        functools.partial(_ode_body, num_steps=10),
        grid=grid,
        in_specs=[
            pl.BlockSpec((block, in_c), lambda i: (i, 0)),
            pl.BlockSpec((in_c, hid), full),
            pl.BlockSpec((1, hid), full),
            pl.BlockSpec((hid, hid), full),
            pl.BlockSpec((1, hid), full),
            pl.BlockSpec((hid, out_c), full),
            pl.BlockSpec((1, out_c), full),
        ],
        out_specs=pl.BlockSpec((block, out_c), lambda i: (i, 0)),
        out_shape=jax.ShapeDtypeStruct((n, out_c), jnp.float32),
    )(x, W1.T, b1.reshape(1, hid), W2.T, b2.reshape(1, hid),
      W3.T, b3.reshape(1, out_c))
    return out
